# Initial kernel scaffold; baseline (speedup 1.0000x reference)
import jax
import jax.numpy as jnp
from jax.experimental import pallas as pl


def kernel(x, edge_index, batch, atom_emb, conv_W0, conv_b0, conv_Wm, conv_bm, conv_Wl, conv_bl, conv_g, conv_beta, conv_t, ln_g, ln_b, lin_W, lin_b):
    raise NotImplementedError("write your pallas kernel here")


# speedup vs baseline: 5.3434x; 5.3434x over previous
import functools

import jax
import jax.numpy as jnp
from jax import lax
from jax.experimental import pallas as pl
from jax.experimental.pallas import tpu as pltpu
from jax.experimental.pallas import tpu_sc as plsc

N = 10000
H = 256
EXP = 512
L = 4
NG = 128
NT = 128

NP_ = 10240
TILE = 1024
GRID = NP_ // TILE
NSUB = 16
CHUNK = 128
RPS = NP_ // NSUB

_f32 = jnp.float32


def _dot_bf16(x, w):
    return jnp.dot(x.astype(jnp.bfloat16), w.astype(jnp.bfloat16),
                   preferred_element_type=_f32)


def _row_mask(i):
    rows = lax.broadcasted_iota(jnp.int32, (TILE, 1), 0) + i * TILE
    return rows < N


def _bn_apply(x, stats_ref, g_ref, b_ref):
    mu = stats_ref[0, :] / N
    var = stats_ref[1, :] / N - mu * mu
    s = jnp.sqrt(var + 1e-5)
    return (x - mu[None, :]) / s[None, :] * g_ref[0, :][None, :] + b_ref[0, :][None, :]


def _accum_stats(i, y, acc_ref, stats_out_ref):
    @pl.when(i == 0)
    def _():
        acc_ref[...] = jnp.zeros_like(acc_ref)

    ym = jnp.where(_row_mask(i), y, 0.0)
    acc_ref[0, :] += jnp.sum(ym, axis=0)
    acc_ref[1, :] += jnp.sum(ym * ym, axis=0)

    @pl.when(i == GRID - 1)
    def _():
        stats_out_ref[...] = acc_ref[...]


def _prep_write(h, t, h_out_ref, a_ref):
    msg = jax.nn.relu(h) + 1e-7
    p = jnp.exp(msg * t)
    pq = p * msg
    h_out_ref[...] = h
    a_ref[0] = p[:, :128]
    a_ref[1] = p[:, 128:]
    a_ref[2] = pq[:, :128]
    a_ref[3] = pq[:, 128:]


def _full(shape):
    return pl.BlockSpec(shape, lambda i: tuple(0 for _ in shape))


_TC_PARAMS = pltpu.CompilerParams(dimension_semantics=("arbitrary",))


def _emb_prep_kernel(x_ref, e0_ref, e1_ref, t_ref, h_ref, a_ref):
    h = jnp.zeros((TILE, H), _f32)
    for f in range(9):
        cond = x_ref[:, f:f + 1] == 1
        row = jnp.where(cond, e1_ref[f:f + 1, :], e0_ref[f:f + 1, :])
        h = h + row
    _prep_write(h, t_ref[0, 0], h_ref, a_ref)


def _emb_prep(xf, e0, e1, t):
    return pl.pallas_call(
        _emb_prep_kernel,
        grid=(GRID,),
        in_specs=[
            pl.BlockSpec((TILE, 16), lambda i: (i, 0)),
            _full((9, H)),
            _full((9, H)),
            _full((1, 1)),
        ],
        out_specs=[
            pl.BlockSpec((TILE, H), lambda i: (i, 0)),
            pl.BlockSpec((4, TILE, 128), lambda i: (0, i, 0)),
        ],
        out_shape=[
            jax.ShapeDtypeStruct((NP_, H), _f32),
            jax.ShapeDtypeStruct((4, NP_, 128), _f32),
        ],
        compiler_params=_TC_PARAMS,
    )(xf, e0, e1, t)


def _prep_kernel(h_ref, stats_ref, g_ref, b_ref, t_ref, u_ref, a_ref):
    u = jax.nn.relu(_bn_apply(h_ref[...], stats_ref, g_ref, b_ref))
    _prep_write(u, t_ref[0, 0], u_ref, a_ref)


def _prep(h, stats, g, b, t):
    return pl.pallas_call(
        _prep_kernel,
        grid=(GRID,),
        in_specs=[
            pl.BlockSpec((TILE, H), lambda i: (i, 0)),
            _full((2, H)),
            _full((1, H)),
            _full((1, H)),
            _full((1, 1)),
        ],
        out_specs=[
            pl.BlockSpec((TILE, H), lambda i: (i, 0)),
            pl.BlockSpec((4, TILE, 128), lambda i: (0, i, 0)),
        ],
        out_shape=[
            jax.ShapeDtypeStruct((NP_, H), _f32),
            jax.ShapeDtypeStruct((4, NP_, 128), _f32),
        ],
        compiler_params=_TC_PARAMS,
    )(h, stats, g, b, t)


def _mlp_in_kernel(b_ref, u_ref, w_ref, bias_ref, y_ref, stats_ref, acc_ref):
    i = pl.program_id(0)
    den = jnp.concatenate([b_ref[0], b_ref[1]], axis=1)
    num = jnp.concatenate([b_ref[2], b_ref[3]], axis=1)
    y0 = num / (den + 1e-16) + u_ref[...]
    y = _dot_bf16(y0, w_ref[...]) + bias_ref[0, :][None, :]
    y_ref[...] = y
    _accum_stats(i, y, acc_ref, stats_ref)


def _mlp_in(bmsg, u, w, bias):
    return pl.pallas_call(
        _mlp_in_kernel,
        grid=(GRID,),
        in_specs=[
            pl.BlockSpec((4, TILE, 128), lambda i: (0, i, 0)),
            pl.BlockSpec((TILE, H), lambda i: (i, 0)),
            _full((H, EXP)),
            _full((1, EXP)),
        ],
        out_specs=[
            pl.BlockSpec((TILE, EXP), lambda i: (i, 0)),
            _full((2, EXP)),
        ],
        out_shape=[
            jax.ShapeDtypeStruct((NP_, EXP), _f32),
            jax.ShapeDtypeStruct((2, EXP), _f32),
        ],
        scratch_shapes=[pltpu.VMEM((2, EXP), _f32)],
        compiler_params=_TC_PARAMS,
    )(bmsg, u, w, bias)


def _unit_kernel(y_ref, stats_in_ref, g_ref, b_ref, w_ref, bias_ref,
                 z_ref, stats_ref, acc_ref):
    i = pl.program_id(0)
    xh = jax.nn.relu(_bn_apply(y_ref[...], stats_in_ref, g_ref, b_ref))
    z = _dot_bf16(xh, w_ref[...]) + bias_ref[0, :][None, :]
    z_ref[...] = z
    _accum_stats(i, z, acc_ref, stats_ref)


def _unit(y, stats, g, b, w, bias):
    return pl.pallas_call(
        _unit_kernel,
        grid=(GRID,),
        in_specs=[
            pl.BlockSpec((TILE, EXP), lambda i: (i, 0)),
            _full((2, EXP)),
            _full((1, EXP)),
            _full((1, EXP)),
            _full((EXP, EXP)),
            _full((1, EXP)),
        ],
        out_specs=[
            pl.BlockSpec((TILE, EXP), lambda i: (i, 0)),
            _full((2, EXP)),
        ],
        out_shape=[
            jax.ShapeDtypeStruct((NP_, EXP), _f32),
            jax.ShapeDtypeStruct((2, EXP), _f32),
        ],
        scratch_shapes=[pltpu.VMEM((2, EXP), _f32)],
        compiler_params=_TC_PARAMS,
    )(y, stats, g, b, w, bias)


def _final_kernel_res(y_ref, stats_in_ref, g_ref, b_ref, w_ref, bias_ref,
                      res_ref, h_ref, stats_ref, acc_ref):
    i = pl.program_id(0)
    xh = jax.nn.relu(_bn_apply(y_ref[...], stats_in_ref, g_ref, b_ref))
    h = _dot_bf16(xh, w_ref[...]) + bias_ref[0, :][None, :]
    h = h + res_ref[...]
    h_ref[...] = h
    _accum_stats(i, h, acc_ref, stats_ref)


def _final_kernel_nores(y_ref, stats_in_ref, g_ref, b_ref, w_ref, bias_ref,
                        h_ref, stats_ref, acc_ref):
    i = pl.program_id(0)
    xh = jax.nn.relu(_bn_apply(y_ref[...], stats_in_ref, g_ref, b_ref))
    h = _dot_bf16(xh, w_ref[...]) + bias_ref[0, :][None, :]
    h_ref[...] = h
    _accum_stats(i, h, acc_ref, stats_ref)


def _final(y, stats, g, b, w, bias, res):
    in_specs = [
        pl.BlockSpec((TILE, EXP), lambda i: (i, 0)),
        _full((2, EXP)),
        _full((1, EXP)),
        _full((1, EXP)),
        _full((EXP, H)),
        _full((1, H)),
    ]
    args = [y, stats, g, b, w, bias]
    if res is not None:
        in_specs.append(pl.BlockSpec((TILE, H), lambda i: (i, 0)))
        args.append(res)
        body = _final_kernel_res
    else:
        body = _final_kernel_nores
    return pl.pallas_call(
        body,
        grid=(GRID,),
        in_specs=in_specs,
        out_specs=[
            pl.BlockSpec((TILE, H), lambda i: (i, 0)),
            _full((2, H)),
        ],
        out_shape=[
            jax.ShapeDtypeStruct((NP_, H), _f32),
            jax.ShapeDtypeStruct((2, H), _f32),
        ],
        scratch_shapes=[pltpu.VMEM((2, H), _f32)],
        compiler_params=_TC_PARAMS,
    )(*args)


def _pool_kernel(h_ref, stats_ref, g_ref, b_ref, batch_ref, w_ref, bias_ref,
                 out_ref, accp_ref, accc_ref):
    i = pl.program_id(0)

    @pl.when(i == 0)
    def _():
        accp_ref[...] = jnp.zeros_like(accp_ref)
        accc_ref[...] = jnp.zeros_like(accc_ref)

    hb = jax.nn.relu(_bn_apply(h_ref[...], stats_ref, g_ref, b_ref))
    groups = lax.broadcasted_iota(jnp.int32, (TILE, NG), 1)
    p = (batch_ref[...] == groups).astype(_f32)
    dn = (((0,), (0,)), ((), ()))
    accp_ref[...] += lax.dot_general(p, hb, dn, preferred_element_type=_f32, precision=lax.Precision.HIGHEST)
    ones = jnp.ones((TILE, 1), _f32)
    accc_ref[...] += lax.dot_general(p, ones, dn, preferred_element_type=_f32, precision=lax.Precision.HIGHEST)

    @pl.when(i == GRID - 1)
    def _():
        pooled = accp_ref[...] / jnp.maximum(accc_ref[...], 1.0)
        out_ref[...] = _dot_bf16(pooled, w_ref[...]) + bias_ref[0, :][None, :]


def _pool(h, stats, g, b, batchp, w, bias):
    return pl.pallas_call(
        _pool_kernel,
        grid=(GRID,),
        in_specs=[
            pl.BlockSpec((TILE, H), lambda i: (i, 0)),
            _full((2, H)),
            _full((1, H)),
            _full((1, H)),
            pl.BlockSpec((TILE, 1), lambda i: (i, 0)),
            _full((H, NT)),
            _full((1, NT)),
        ],
        out_specs=_full((NG, NT)),
        out_shape=jax.ShapeDtypeStruct((NG, NT), _f32),
        scratch_shapes=[pltpu.VMEM((NG, H), _f32), pltpu.VMEM((NG, 1), _f32)],
        compiler_params=_TC_PARAMS,
    )(h, stats, g, b, batchp, w, bias)


def _make_spmm(num_chunks):
    mesh = plsc.VectorSubcoreMesh(core_axis_name="c", subcore_axis_name="s")
    j_cnt = num_chunks

    @functools.partial(
        pl.kernel,
        out_type=jax.ShapeDtypeStruct((4 * NP_, 128), _f32),
        mesh=mesh,
        scratch_types=[
            pltpu.VMEM((j_cnt, CHUNK), jnp.int32),
            pltpu.VMEM((CHUNK,), jnp.int32),
            pltpu.VMEM((j_cnt, CHUNK), jnp.int32),
            pltpu.VMEM((CHUNK, 128), _f32),
            pltpu.VMEM_SHARED((NP_, 128), _f32),
            pltpu.SemaphoreType.DMA,
        ],
    )
    def spmm(a_hbm, src_hbm, dst_hbm, zero_hbm, out_hbm,
             src_v, srck_v, dst_v, gbuf, acc, sem):
        c = lax.axis_index("c")
        s = lax.axis_index("s")
        pltpu.sync_copy(src_hbm.at[s], src_v)
        pltpu.sync_copy(dst_hbm.at[s], dst_v)
        for k in range(2):
            cc = 2 * c + k
            off = cc * NP_

            pltpu.sync_copy(zero_hbm, gbuf)
            for z in range(RPS // CHUNK):
                pltpu.sync_copy(gbuf, acc.at[pl.ds(s * RPS + z * CHUNK, CHUNK)])
            plsc.subcore_barrier()

            def edge_body(j, carry):
                for uu in range(CHUNK // 16):
                    sl = pl.ds(uu * 16, 16)
                    srck_v[sl] = src_v[j, sl] + off
                pltpu.async_copy(a_hbm.at[srck_v], gbuf, sem).wait()
                pltpu.sync_copy(gbuf, acc.at[dst_v.at[j]], add=True)
                return carry

            lax.fori_loop(0, j_cnt, edge_body, 0)
            plsc.subcore_barrier()

            for z in range(RPS // CHUNK):
                r0 = s * RPS + z * CHUNK
                pltpu.sync_copy(acc.at[pl.ds(r0, CHUNK)], gbuf)
                pltpu.sync_copy(gbuf, out_hbm.at[pl.ds(off + r0, CHUNK)])
            plsc.subcore_barrier()

    return spmm


def _spmm_apply(a, src_r, dst_r, zeros128, num_chunks):
    return _make_spmm(num_chunks)(a.reshape(4 * NP_, 128), src_r, dst_r, zeros128)


def kernel(x, edge_index, batch, atom_emb, conv_W0, conv_b0, conv_Wm, conv_bm,
           conv_Wl, conv_bl, conv_g, conv_beta, conv_t, ln_g, ln_b, lin_W, lin_b):
    e_total = edge_index.shape[1]
    j_cnt = -(-e_total // (NSUB * CHUNK))
    e_pad = j_cnt * NSUB * CHUNK

    xi = jnp.pad(x.astype(jnp.int32), ((0, NP_ - N), (0, 7)))
    e0 = atom_emb[:, 0, :]
    e1 = atom_emb[:, 1, :]

    src = jnp.pad(edge_index[0].astype(jnp.int32), (0, e_pad - e_total))
    dst = jnp.pad(edge_index[1].astype(jnp.int32), (0, e_pad - e_total),
                  constant_values=N)
    src_r = src.reshape(NSUB, j_cnt, CHUNK)
    dst_r = dst.reshape(NSUB, j_cnt, CHUNK)
    zeros128 = jnp.zeros((CHUNK, 128), _f32)
    batchp = jnp.pad(batch.astype(jnp.int32), (0, NP_ - N),
                     constant_values=NG).reshape(NP_, 1)

    h = None
    stats_h = None
    for l in range(L):
        t = conv_t[l].reshape(1, 1)
        if l == 0:
            u, a = _emb_prep(xi, e0, e1, t)
            res = None
        else:
            u, a = _prep(h, stats_h, ln_g[l][None], ln_b[l][None], t)
            res = h
        bmsg = _spmm_apply(a, src_r, dst_r, zeros128, j_cnt).reshape(4, NP_, 128)
        y, st = _mlp_in(bmsg, u, conv_W0[l], conv_b0[l][None])
        for i in range(5):
            y, st = _unit(y, st, conv_g[l, i][None], conv_beta[l, i][None],
                          conv_Wm[l, i], conv_bm[l, i][None])
        h, stats_h = _final(y, st, conv_g[l, 5][None], conv_beta[l, 5][None],
                            conv_Wl[l], conv_bl[l][None], res)

    return _pool(h, stats_h, ln_g[0][None], ln_b[0][None], batchp,
                 lin_W, lin_b[None])
